# Initial kernel scaffold; baseline (speedup 1.0000x reference)
import jax
import jax.numpy as jnp
from jax.experimental import pallas as pl


def kernel(idx, embedding):
    raise NotImplementedError("write your pallas kernel here")


# speedup vs baseline: 1.1050x; 1.1050x over previous
import functools

import jax
import jax.numpy as jnp
from jax import lax
from jax.experimental import pallas as pl
from jax.experimental.pallas import tpu as pltpu
from jax.experimental.pallas import tpu_sc as plsc

EMBED_DIM = 32
_NC, _NS = 2, 16
_NW = _NC * _NS

_B = 16384 * 50
_B_PER_W = _B // _NW
_CHUNK = 1280
_NCHUNK = _B_PER_W // _CHUNK

_mesh = plsc.VectorSubcoreMesh(core_axis_name="c", subcore_axis_name="s")


@functools.partial(
    pl.kernel,
    out_type=jax.ShapeDtypeStruct((_B, EMBED_DIM), jnp.float32),
    mesh=_mesh,
    compiler_params=pltpu.CompilerParams(use_tc_tiling_on_sc=False),
    scratch_types=[
        pltpu.VMEM((_B_PER_W,), jnp.int32),
        pltpu.VMEM((_CHUNK, EMBED_DIM), jnp.float32),
        pltpu.SemaphoreType.DMA,
    ],
)
def _gather(idx_hbm, table_hbm, out_hbm, idx_v, rows_v, sem):
    wid = lax.axis_index("s") * _NC + lax.axis_index("c")
    base = wid * _B_PER_W
    pltpu.sync_copy(idx_hbm.at[pl.ds(base, _B_PER_W)], idx_v)

    def body(g, carry):
        off = g * _CHUNK
        pltpu.async_copy(
            table_hbm.at[idx_v.at[pl.ds(off, _CHUNK)]], rows_v, sem
        ).wait()
        pltpu.sync_copy(rows_v, out_hbm.at[pl.ds(base + off, _CHUNK)])
        return carry

    lax.fori_loop(0, _NCHUNK, body, 0)


def kernel(idx, embedding):
    idx_flat = idx.reshape(_B).astype(jnp.int32)
    out = _gather(idx_flat, embedding)
    return out.reshape(idx.shape + (EMBED_DIM,))


# speedup vs baseline: 1.1111x; 1.0055x over previous
import functools

import jax
import jax.numpy as jnp
from jax import lax
from jax.experimental import pallas as pl
from jax.experimental.pallas import tpu as pltpu
from jax.experimental.pallas import tpu_sc as plsc

EMBED_DIM = 32
_NC, _NS = 2, 16
_NW = _NC * _NS

_B = 16384 * 50
_B_PER_W = _B // _NW
_CHUNK = 1280
_NCHUNK = _B_PER_W // _CHUNK

_mesh = plsc.VectorSubcoreMesh(core_axis_name="c", subcore_axis_name="s")


@functools.partial(
    pl.kernel,
    out_type=jax.ShapeDtypeStruct((_B, EMBED_DIM), jnp.float32),
    mesh=_mesh,
    compiler_params=pltpu.CompilerParams(use_tc_tiling_on_sc=False),
    scratch_types=[
        pltpu.VMEM((_B_PER_W,), jnp.int32),
        pltpu.VMEM((2, _CHUNK, EMBED_DIM), jnp.float32),
        pltpu.SemaphoreType.DMA,
        pltpu.SemaphoreType.DMA,
    ],
)
def _gather(idx_hbm, table_hbm, out_hbm, idx_v, rows_v, sem0, sem1):
    wid = lax.axis_index("s") * _NC + lax.axis_index("c")
    base = wid * _B_PER_W
    pltpu.sync_copy(idx_hbm.at[pl.ds(base, _B_PER_W)], idx_v)

    def start(g, buf, sem):
        off = g * _CHUNK
        pltpu.async_copy(
            table_hbm.at[idx_v.at[pl.ds(off, _CHUNK)]], rows_v.at[buf], sem
        )

    def wait(buf, sem):
        pltpu.make_async_copy(
            table_hbm.at[idx_v.at[pl.ds(0, _CHUNK)]], rows_v.at[buf], sem
        ).wait()

    def flush(g, buf):
        pltpu.sync_copy(rows_v.at[buf], out_hbm.at[pl.ds(base + g * _CHUNK, _CHUNK)])

    start(0, 0, sem0)

    def body(i, carry):
        g = 2 * i
        wait(0, sem0)
        start(g + 1, 1, sem1)
        flush(g, 0)
        wait(1, sem1)

        @pl.when(g + 2 < _NCHUNK)
        def _start_next():
            start(g + 2, 0, sem0)

        flush(g + 1, 1)
        return carry

    lax.fori_loop(0, _NCHUNK // 2, body, 0)


def kernel(idx, embedding):
    idx_flat = idx.reshape(_B).astype(jnp.int32)
    out = _gather(idx_flat, embedding)
    return out.reshape(idx.shape + (EMBED_DIM,))


# speedup vs baseline: 1.6978x; 1.5281x over previous
import functools

import jax
import jax.numpy as jnp
from jax import lax
from jax.experimental import pallas as pl
from jax.experimental.pallas import tpu as pltpu
from jax.experimental.pallas import tpu_sc as plsc

EMBED_DIM = 32
_SEQ = 50
_NSAMPLES = 16384
_NC, _NS = 2, 16
_NW = _NC * _NS
_S_PER_W = _NSAMPLES // _NW
_K = 4
_NBLK = _S_PER_W // _K

_mesh = plsc.VectorSubcoreMesh(core_axis_name="c", subcore_axis_name="s")


@functools.partial(
    pl.kernel,
    out_type=jax.ShapeDtypeStruct((_NSAMPLES, _SEQ, EMBED_DIM), jnp.float32),
    mesh=_mesh,
    compiler_params=pltpu.CompilerParams(use_tc_tiling_on_sc=False),
    scratch_types=[
        pltpu.VMEM((_S_PER_W, _SEQ), jnp.int32),
        pltpu.VMEM((2, _K, _SEQ, EMBED_DIM), jnp.float32),
        pltpu.SemaphoreType.DMA,
        pltpu.SemaphoreType.DMA,
    ],
)
def _gather(idx_hbm, table_hbm, out_hbm, idx_v, rows_v, sem0, sem1):
    wid = lax.axis_index("s") * _NC + lax.axis_index("c")
    sample0 = wid * _S_PER_W
    pltpu.sync_copy(idx_hbm.at[pl.ds(sample0, _S_PER_W)], idx_v)

    def start(blk, buf, sem):
        for j in range(_K):
            pltpu.async_copy(
                table_hbm.at[idx_v.at[blk * _K + j]], rows_v.at[buf, j], sem
            )

    def wait(buf, sem):
        for j in range(_K):
            pltpu.make_async_copy(
                table_hbm.at[idx_v.at[0]], rows_v.at[buf, j], sem
            ).wait()

    def flush(blk, buf):
        pltpu.sync_copy(
            rows_v.at[buf], out_hbm.at[pl.ds(sample0 + blk * _K, _K)]
        )

    start(0, 0, sem0)

    def body(i, carry):
        blk = 2 * i
        wait(0, sem0)
        start(blk + 1, 1, sem1)
        flush(blk, 0)
        wait(1, sem1)

        @pl.when(blk + 2 < _NBLK)
        def _start_next():
            start(blk + 2, 0, sem0)

        flush(blk + 1, 1)
        return carry

    lax.fori_loop(0, _NBLK // 2, body, 0)


def kernel(idx, embedding):
    return _gather(idx.astype(jnp.int32), embedding)


# speedup vs baseline: 1.7625x; 1.0381x over previous
import functools

import jax
import jax.numpy as jnp
from jax import lax
from jax.experimental import pallas as pl
from jax.experimental.pallas import tpu as pltpu
from jax.experimental.pallas import tpu_sc as plsc

EMBED_DIM = 32
_SEQ = 50
_NSAMPLES = 16384
_NC, _NS = 2, 16
_NW = _NC * _NS
_S_PER_W = _NSAMPLES // _NW
_K = 8
_NBLK = _S_PER_W // _K

_mesh = plsc.VectorSubcoreMesh(core_axis_name="c", subcore_axis_name="s")


@functools.partial(
    pl.kernel,
    out_type=jax.ShapeDtypeStruct((_NSAMPLES, _SEQ, EMBED_DIM), jnp.float32),
    mesh=_mesh,
    compiler_params=pltpu.CompilerParams(use_tc_tiling_on_sc=False),
    scratch_types=[
        pltpu.VMEM((_S_PER_W, _SEQ), jnp.int32),
        pltpu.VMEM((2, _K, _SEQ, EMBED_DIM), jnp.float32),
        pltpu.SemaphoreType.DMA,
        pltpu.SemaphoreType.DMA,
    ],
)
def _gather(idx_hbm, table_hbm, out_hbm, idx_v, rows_v, sem0, sem1):
    wid = lax.axis_index("s") * _NC + lax.axis_index("c")
    sample0 = wid * _S_PER_W
    pltpu.sync_copy(idx_hbm.at[pl.ds(sample0, _S_PER_W)], idx_v)

    def start(blk, buf, sem):
        for j in range(_K):
            pltpu.async_copy(
                table_hbm.at[idx_v.at[blk * _K + j]], rows_v.at[buf, j], sem
            )

    def wait(buf, sem):
        for j in range(_K):
            pltpu.make_async_copy(
                table_hbm.at[idx_v.at[0]], rows_v.at[buf, j], sem
            ).wait()

    def flush(blk, buf):
        pltpu.sync_copy(
            rows_v.at[buf], out_hbm.at[pl.ds(sample0 + blk * _K, _K)]
        )

    start(0, 0, sem0)

    def body(i, carry):
        blk = 2 * i
        wait(0, sem0)
        start(blk + 1, 1, sem1)
        flush(blk, 0)
        wait(1, sem1)

        @pl.when(blk + 2 < _NBLK)
        def _start_next():
            start(blk + 2, 0, sem0)

        flush(blk + 1, 1)
        return carry

    lax.fori_loop(0, _NBLK // 2, body, 0)


def kernel(idx, embedding):
    return _gather(idx.astype(jnp.int32), embedding)


# speedup vs baseline: 1.7939x; 1.0178x over previous
import functools

import jax
import jax.numpy as jnp
from jax import lax
from jax.experimental import pallas as pl
from jax.experimental.pallas import tpu as pltpu
from jax.experimental.pallas import tpu_sc as plsc

EMBED_DIM = 32
_SEQ = 50
_NSAMPLES = 16384
_NC, _NS = 2, 16
_NW = _NC * _NS
_S_PER_W = _NSAMPLES // _NW
_K = 16
_NBLK = _S_PER_W // _K

_mesh = plsc.VectorSubcoreMesh(core_axis_name="c", subcore_axis_name="s")


@functools.partial(
    pl.kernel,
    out_type=jax.ShapeDtypeStruct((_NSAMPLES, _SEQ, EMBED_DIM), jnp.float32),
    mesh=_mesh,
    compiler_params=pltpu.CompilerParams(use_tc_tiling_on_sc=False),
    scratch_types=[
        pltpu.VMEM((_S_PER_W, _SEQ), jnp.int32),
        pltpu.VMEM((2, _K, _SEQ, EMBED_DIM), jnp.float32),
        pltpu.SemaphoreType.DMA,
        pltpu.SemaphoreType.DMA,
    ],
)
def _gather(idx_hbm, table_hbm, out_hbm, idx_v, rows_v, sem0, sem1):
    wid = lax.axis_index("s") * _NC + lax.axis_index("c")
    sample0 = wid * _S_PER_W
    pltpu.sync_copy(idx_hbm.at[pl.ds(sample0, _S_PER_W)], idx_v)

    def start(blk, buf, sem):
        for j in range(_K):
            pltpu.async_copy(
                table_hbm.at[idx_v.at[blk * _K + j]], rows_v.at[buf, j], sem
            )

    def wait(buf, sem):
        for j in range(_K):
            pltpu.make_async_copy(
                table_hbm.at[idx_v.at[0]], rows_v.at[buf, j], sem
            ).wait()

    def flush(blk, buf):
        pltpu.sync_copy(
            rows_v.at[buf], out_hbm.at[pl.ds(sample0 + blk * _K, _K)]
        )

    start(0, 0, sem0)

    def body(i, carry):
        blk = 2 * i
        wait(0, sem0)
        start(blk + 1, 1, sem1)
        flush(blk, 0)
        wait(1, sem1)

        @pl.when(blk + 2 < _NBLK)
        def _start_next():
            start(blk + 2, 0, sem0)

        flush(blk + 1, 1)
        return carry

    lax.fori_loop(0, _NBLK // 2, body, 0)


def kernel(idx, embedding):
    return _gather(idx.astype(jnp.int32), embedding)
